# Initial kernel scaffold; baseline (speedup 1.0000x reference)
import jax
import jax.numpy as jnp
from jax.experimental import pallas as pl


def kernel(x, edge_index, Wp, bp, W_rel1, b_rel1, W_root1, W_rel2, b_rel2, W_root2, ln_w, ln_b, Wf, bf):
    raise NotImplementedError("write your pallas kernel here")


# speedup vs baseline: 3.7783x; 3.7783x over previous
import functools

import jax
import jax.numpy as jnp
from jax import lax
from jax.experimental import pallas as pl
from jax.experimental.pallas import tpu as pltpu
from jax.experimental.pallas import tpu_sc as plsc

D = 128
NW = 32
CHUNK = 128
ROW_BLOCK = 1000


@functools.partial(jax.jit, static_argnames=("n_pad", "ep"))
def _sc_segment_sum(h, src, dst, zeros, *, n_pad, ep):
    epw = ep // NW
    n_chunks = epw // CHUNK
    rows_per_tile = n_pad // 16

    mesh = plsc.VectorSubcoreMesh(core_axis_name="c", subcore_axis_name="s")

    @functools.partial(
        pl.kernel,
        mesh=mesh,
        out_type=jax.ShapeDtypeStruct((2, n_pad, D), jnp.float32),
        scratch_types=[
            pltpu.VMEM((CHUNK,), jnp.int32),
            pltpu.VMEM((CHUNK,), jnp.int32),
            pltpu.VMEM((CHUNK, D), jnp.float32),
            pltpu.VMEM_SHARED((n_pad, D), jnp.float32),
            pltpu.SemaphoreType.DMA,
        ],
    )
    def agg(h_hbm, src_hbm, dst_hbm, zero_hbm, out_hbm,
            src_v, dst_v, rows_v, acc_sh, sem):
        c = lax.axis_index("c")
        s = lax.axis_index("s")
        wid = c * 16 + s
        r0 = s * rows_per_tile
        pltpu.sync_copy(zero_hbm.at[pl.ds(r0, rows_per_tile)],
                        acc_sh.at[pl.ds(r0, rows_per_tile)])
        plsc.subcore_barrier()

        base0 = wid * epw

        def body(k, carry):
            base = base0 + k * CHUNK
            pltpu.sync_copy(src_hbm.at[pl.ds(base, CHUNK)], src_v)
            pltpu.sync_copy(dst_hbm.at[pl.ds(base, CHUNK)], dst_v)
            pltpu.async_copy(h_hbm.at[src_v], rows_v, sem).wait()
            pltpu.sync_copy(rows_v, acc_sh.at[dst_v], add=True)
            return carry

        lax.fori_loop(0, n_chunks, body, 0)
        plsc.subcore_barrier()
        pltpu.sync_copy(acc_sh.at[pl.ds(r0, rows_per_tile)],
                        out_hbm.at[c].at[pl.ds(r0, rows_per_tile)])

    return agg(h, src, dst, zeros)


def _proj_body(x_ref, w_ref, b_ref, o_ref):
    o_ref[:] = jnp.dot(x_ref[:], w_ref[:],
                       preferred_element_type=jnp.float32) + b_ref[:]


def _update_body(p0_ref, p1_ref, h_ref, wr_ref, br_ref, wt_ref, o_ref):
    agg = p0_ref[:] + p1_ref[:]
    o_ref[:] = jnp.maximum(
        jnp.dot(agg, wr_ref[:], preferred_element_type=jnp.float32)
        + br_ref[:]
        + jnp.dot(h_ref[:], wt_ref[:], preferred_element_type=jnp.float32),
        0.0)


def _final_body(p0_ref, p1_ref, h_ref, wr_ref, br_ref, wt_ref,
                lnw_ref, lnb_ref, wf_ref, bf_ref, o_ref):
    agg = p0_ref[:] + p1_ref[:]
    h2 = jnp.maximum(
        jnp.dot(agg, wr_ref[:], preferred_element_type=jnp.float32)
        + br_ref[:]
        + jnp.dot(h_ref[:], wt_ref[:], preferred_element_type=jnp.float32),
        0.0)
    mu = jnp.mean(h2, axis=-1, keepdims=True)
    cent = h2 - mu
    var = jnp.mean(cent * cent, axis=-1, keepdims=True)
    normed = cent * lax.rsqrt(var + 1e-5) * lnw_ref[:] + lnb_ref[:]
    o_ref[:] = jnp.dot(normed, wf_ref[:],
                       preferred_element_type=jnp.float32) + bf_ref[:]


def _row_spec():
    return pl.BlockSpec((ROW_BLOCK, D), lambda i: (i, 0))


def _full_spec():
    return pl.BlockSpec((D, D), lambda i: (0, 0))


def _vec_spec():
    return pl.BlockSpec((1, D), lambda i: (0, 0))


def _tc_proj(x, w_t, b):
    n = x.shape[0]
    return pl.pallas_call(
        _proj_body,
        grid=(n // ROW_BLOCK,),
        in_specs=[_row_spec(), _full_spec(), _vec_spec()],
        out_specs=_row_spec(),
        out_shape=jax.ShapeDtypeStruct((n, D), jnp.float32),
    )(x, w_t, b)


def _tc_update(p0, p1, h, wr_t, br, wt_t):
    n = h.shape[0]
    return pl.pallas_call(
        _update_body,
        grid=(n // ROW_BLOCK,),
        in_specs=[_row_spec(), _row_spec(), _row_spec(),
                  _full_spec(), _vec_spec(), _full_spec()],
        out_specs=_row_spec(),
        out_shape=jax.ShapeDtypeStruct((n, D), jnp.float32),
    )(p0, p1, h, wr_t, br, wt_t)


def _tc_final(p0, p1, h, wr_t, br, wt_t, lnw, lnb, wf_t, bf):
    n = h.shape[0]
    return pl.pallas_call(
        _final_body,
        grid=(n // ROW_BLOCK,),
        in_specs=[_row_spec(), _row_spec(), _row_spec(),
                  _full_spec(), _vec_spec(), _full_spec(),
                  _vec_spec(), _vec_spec(), _full_spec(), _vec_spec()],
        out_specs=_row_spec(),
        out_shape=jax.ShapeDtypeStruct((n, D), jnp.float32),
    )(p0, p1, h, wr_t, br, wt_t, lnw, lnb, wf_t, bf)


def kernel(x, edge_index, Wp, bp, W_rel1, b_rel1, W_root1,
           W_rel2, b_rel2, W_root2, ln_w, ln_b, Wf, bf):
    n = x.shape[0]
    e = edge_index.shape[1]

    n_pad = -(-(n + 1) // 128) * 128
    ep = -(-e // (NW * CHUNK)) * (NW * CHUNK)

    src = edge_index[0].astype(jnp.int32)
    dst = edge_index[1].astype(jnp.int32)
    pad_e = ep - e
    if pad_e:
        src = jnp.concatenate([src, jnp.zeros((pad_e,), jnp.int32)])
        dst = jnp.concatenate([dst, jnp.full((pad_e,), n, jnp.int32)])
    zeros = jnp.zeros((n_pad, D), jnp.float32)

    wp_t = Wp.T
    wr1_t, wt1_t = W_rel1.T, W_root1.T
    wr2_t, wt2_t = W_rel2.T, W_root2.T
    wf_t = Wf.T
    bp2 = bp.reshape(1, D)
    br1 = b_rel1.reshape(1, D)
    br2 = b_rel2.reshape(1, D)
    lnw2 = ln_w.reshape(1, D)
    lnb2 = ln_b.reshape(1, D)
    bf2 = bf.reshape(1, D)

    h0 = _tc_proj(x, wp_t, bp2)

    part1 = _sc_segment_sum(h0, src, dst, zeros, n_pad=n_pad, ep=ep)
    h1 = _tc_update(part1[0, :n], part1[1, :n], h0, wr1_t, br1, wt1_t)

    part2 = _sc_segment_sum(h1, src, dst, zeros, n_pad=n_pad, ep=ep)
    out = _tc_final(part2[0, :n], part2[1, :n], h1, wr2_t, br2, wt2_t,
                    lnw2, lnb2, wf_t, bf2)
    return out


# speedup vs baseline: 4.8782x; 1.2911x over previous
import functools

import jax
import jax.numpy as jnp
from jax import lax
from jax.experimental import pallas as pl
from jax.experimental.pallas import tpu as pltpu
from jax.experimental.pallas import tpu_sc as plsc

D = 128
NW = 32
CHUNK = 128
ROW_BLOCK = 1000


@functools.partial(jax.jit, static_argnames=("n_pad", "ep"))
def _sc_segment_sum(h, src, dst, zeros, *, n_pad, ep):
    epw = ep // NW
    n_chunks = epw // CHUNK
    rows_per_tile = n_pad // 16

    mesh = plsc.VectorSubcoreMesh(core_axis_name="c", subcore_axis_name="s")

    @functools.partial(
        pl.kernel,
        mesh=mesh,
        out_type=jax.ShapeDtypeStruct((2, n_pad, D), jnp.float32),
        scratch_types=[
            pltpu.VMEM((2, 2, CHUNK), jnp.int32),
            pltpu.VMEM((2, CHUNK, D), jnp.float32),
            pltpu.VMEM_SHARED((n_pad, D), jnp.float32),
            pltpu.SemaphoreType.DMA,
            pltpu.SemaphoreType.DMA,
            pltpu.SemaphoreType.DMA,
            pltpu.SemaphoreType.DMA,
        ],
    )
    def agg(h_hbm, idx_hbm, zero_hbm, out_hbm,
            idx_v, rows_v, acc_sh, semi0, semi1, semg0, semg1):
        c = lax.axis_index("c")
        s = lax.axis_index("s")
        wid = c * 16 + s
        r0 = s * rows_per_tile
        pltpu.sync_copy(zero_hbm.at[pl.ds(r0, rows_per_tile)],
                        acc_sh.at[pl.ds(r0, rows_per_tile)])
        plsc.subcore_barrier()

        semi = (semi0, semi1)
        semg = (semg0, semg1)

        def idx_load(j, b):
            return pltpu.make_async_copy(idx_hbm.at[wid].at[j],
                                         idx_v.at[b], semi[b])

        def gather(j, b):
            del j
            return pltpu.make_async_copy(h_hbm.at[idx_v.at[b].at[0]],
                                         rows_v.at[b], semg[b])

        idx_load(0, 0).start()
        idx_load(1, 1).start()
        idx_load(0, 0).wait()
        gather(0, 0).start()

        def body(j, carry):
            b = lax.rem(j, 2)
            for bb in range(2):
                @pl.when(b == bb)
                def _():
                    gather(j, bb).wait()

                    @pl.when(j + 1 < n_chunks)
                    def _():
                        idx_load(j + 1, 1 - bb).wait()
                        gather(j + 1, 1 - bb).start()
                    pltpu.sync_copy(rows_v.at[bb],
                                    acc_sh.at[idx_v.at[bb].at[1]],
                                    add=True)

                    @pl.when(j + 2 < n_chunks)
                    def _():
                        idx_load(j + 2, bb).start()
            return carry

        lax.fori_loop(0, n_chunks, body, 0)
        plsc.subcore_barrier()
        pltpu.sync_copy(acc_sh.at[pl.ds(r0, rows_per_tile)],
                        out_hbm.at[c].at[pl.ds(r0, rows_per_tile)])

    idx_all = jnp.stack([src.reshape(NW, n_chunks, CHUNK),
                         dst.reshape(NW, n_chunks, CHUNK)], axis=2)
    return agg(h.astype(jnp.float32), idx_all, zeros)


def _proj_body(x_ref, w_ref, b_ref, o_ref):
    o_ref[:] = jnp.dot(x_ref[:], w_ref[:],
                       preferred_element_type=jnp.float32) + b_ref[:]


def _update_body(p0_ref, p1_ref, h_ref, wr_ref, br_ref, wt_ref, o_ref):
    agg = p0_ref[:] + p1_ref[:]
    o_ref[:] = jnp.maximum(
        jnp.dot(agg, wr_ref[:], preferred_element_type=jnp.float32)
        + br_ref[:]
        + jnp.dot(h_ref[:], wt_ref[:], preferred_element_type=jnp.float32),
        0.0)


def _final_body(p0_ref, p1_ref, h_ref, wr_ref, br_ref, wt_ref,
                lnw_ref, lnb_ref, wf_ref, bf_ref, o_ref):
    agg = p0_ref[:] + p1_ref[:]
    h2 = jnp.maximum(
        jnp.dot(agg, wr_ref[:], preferred_element_type=jnp.float32)
        + br_ref[:]
        + jnp.dot(h_ref[:], wt_ref[:], preferred_element_type=jnp.float32),
        0.0)
    mu = jnp.mean(h2, axis=-1, keepdims=True)
    cent = h2 - mu
    var = jnp.mean(cent * cent, axis=-1, keepdims=True)
    normed = cent * lax.rsqrt(var + 1e-5) * lnw_ref[:] + lnb_ref[:]
    o_ref[:] = jnp.dot(normed, wf_ref[:],
                       preferred_element_type=jnp.float32) + bf_ref[:]


def _row_spec():
    return pl.BlockSpec((ROW_BLOCK, D), lambda i: (i, 0))


def _full_spec():
    return pl.BlockSpec((D, D), lambda i: (0, 0))


def _vec_spec():
    return pl.BlockSpec((1, D), lambda i: (0, 0))


def _tc_proj(x, w_t, b):
    n = x.shape[0]
    return pl.pallas_call(
        _proj_body,
        grid=(n // ROW_BLOCK,),
        in_specs=[_row_spec(), _full_spec(), _vec_spec()],
        out_specs=_row_spec(),
        out_shape=jax.ShapeDtypeStruct((n, D), jnp.float32),
    )(x, w_t, b)


def _tc_update(p0, p1, h, wr_t, br, wt_t):
    n = h.shape[0]
    return pl.pallas_call(
        _update_body,
        grid=(n // ROW_BLOCK,),
        in_specs=[_row_spec(), _row_spec(), _row_spec(),
                  _full_spec(), _vec_spec(), _full_spec()],
        out_specs=_row_spec(),
        out_shape=jax.ShapeDtypeStruct((n, D), jnp.float32),
    )(p0, p1, h, wr_t, br, wt_t)


def _tc_final(p0, p1, h, wr_t, br, wt_t, lnw, lnb, wf_t, bf):
    n = h.shape[0]
    return pl.pallas_call(
        _final_body,
        grid=(n // ROW_BLOCK,),
        in_specs=[_row_spec(), _row_spec(), _row_spec(),
                  _full_spec(), _vec_spec(), _full_spec(),
                  _vec_spec(), _vec_spec(), _full_spec(), _vec_spec()],
        out_specs=_row_spec(),
        out_shape=jax.ShapeDtypeStruct((n, D), jnp.float32),
    )(p0, p1, h, wr_t, br, wt_t, lnw, lnb, wf_t, bf)


def kernel(x, edge_index, Wp, bp, W_rel1, b_rel1, W_root1,
           W_rel2, b_rel2, W_root2, ln_w, ln_b, Wf, bf):
    n = x.shape[0]
    e = edge_index.shape[1]

    n_pad = -(-(n + 1) // 128) * 128
    ep = -(-e // (NW * CHUNK)) * (NW * CHUNK)

    src = edge_index[0].astype(jnp.int32)
    dst = edge_index[1].astype(jnp.int32)
    pad_e = ep - e
    if pad_e:
        src = jnp.concatenate([src, jnp.zeros((pad_e,), jnp.int32)])
        junk = n + (jnp.arange(pad_e, dtype=jnp.int32) % (n_pad - n))
        dst = jnp.concatenate([dst, junk])
    zeros = jnp.zeros((n_pad, D), jnp.float32)

    wp_t = Wp.T
    wr1_t, wt1_t = W_rel1.T, W_root1.T
    wr2_t, wt2_t = W_rel2.T, W_root2.T
    wf_t = Wf.T
    bp2 = bp.reshape(1, D)
    br1 = b_rel1.reshape(1, D)
    br2 = b_rel2.reshape(1, D)
    lnw2 = ln_w.reshape(1, D)
    lnb2 = ln_b.reshape(1, D)
    bf2 = bf.reshape(1, D)

    h0 = _tc_proj(x, wp_t, bp2)

    part1 = _sc_segment_sum(h0, src, dst, zeros, n_pad=n_pad, ep=ep)
    h1 = _tc_update(part1[0, :n], part1[1, :n], h0, wr1_t, br1, wt1_t)

    part2 = _sc_segment_sum(h1, src, dst, zeros, n_pad=n_pad, ep=ep)
    out = _tc_final(part2[0, :n], part2[1, :n], h1, wr2_t, br2, wt2_t,
                    lnw2, lnb2, wf_t, bf2)
    return out


# speedup vs baseline: 10.1495x; 2.0806x over previous
import functools

import jax
import jax.numpy as jnp
from jax import lax
from jax.experimental import pallas as pl
from jax.experimental.pallas import tpu as pltpu
from jax.experimental.pallas import tpu_sc as plsc

D = 128
NW = 32
CHUNK = 128
ROW_BLOCK = 1000


@functools.partial(jax.jit, static_argnames=("n_pad", "ep"))
def _sc_segment_sum(h, src, dst, zeros, *, n_pad, ep):
    epw = ep // NW
    n_chunks = epw // CHUNK
    rows_per_tile = n_pad // 16

    mesh = plsc.VectorSubcoreMesh(core_axis_name="c", subcore_axis_name="s")

    @functools.partial(
        pl.kernel,
        mesh=mesh,
        out_type=jax.ShapeDtypeStruct((2, n_pad, D), jnp.float32),
        scratch_types=[
            pltpu.VMEM((2, 2, CHUNK), jnp.int32),
            pltpu.VMEM((2, CHUNK, D), jnp.float32),
            pltpu.VMEM_SHARED((n_pad, D), jnp.float32),
            pltpu.SemaphoreType.DMA,
            pltpu.SemaphoreType.DMA,
            pltpu.SemaphoreType.DMA,
            pltpu.SemaphoreType.DMA,
        ],
    )
    def agg(h_hbm, idx_hbm, zero_hbm, out_hbm,
            idx_v, rows_v, acc_sh, semi0, semi1, semg0, semg1):
        c = lax.axis_index("c")
        s = lax.axis_index("s")
        wid = c * 16 + s
        r0 = s * rows_per_tile
        pltpu.sync_copy(zero_hbm.at[pl.ds(r0, rows_per_tile)],
                        acc_sh.at[pl.ds(r0, rows_per_tile)])
        plsc.subcore_barrier()

        semi = (semi0, semi1)
        semg = (semg0, semg1)

        def idx_load(j, b):
            return pltpu.make_async_copy(idx_hbm.at[wid].at[j],
                                         idx_v.at[b], semi[b])

        def gather(j, b):
            del j
            return pltpu.make_async_copy(h_hbm.at[idx_v.at[b].at[0]],
                                         rows_v.at[b], semg[b])

        idx_load(0, 0).start()
        idx_load(1, 1).start()
        idx_load(0, 0).wait()
        gather(0, 0).start()

        def body(j, carry):
            b = lax.rem(j, 2)
            for bb in range(2):
                @pl.when(b == bb)
                def _():
                    gather(j, bb).wait()

                    @pl.when(j + 1 < n_chunks)
                    def _():
                        idx_load(j + 1, 1 - bb).wait()
                        gather(j + 1, 1 - bb).start()
                    pltpu.sync_copy(rows_v.at[bb],
                                    acc_sh.at[idx_v.at[bb].at[1]],
                                    add=True)

                    @pl.when(j + 2 < n_chunks)
                    def _():
                        idx_load(j + 2, bb).start()
            return carry

        lax.fori_loop(0, n_chunks, body, 0)
        plsc.subcore_barrier()
        pltpu.sync_copy(acc_sh.at[pl.ds(r0, rows_per_tile)],
                        out_hbm.at[c].at[pl.ds(r0, rows_per_tile)])

    idx_all = jnp.stack([src.reshape(NW, n_chunks, CHUNK),
                         dst.reshape(NW, n_chunks, CHUNK)], axis=2)
    return agg(h.astype(jnp.float32), idx_all, zeros)


def _proj_body(x_ref, w_ref, b_ref, o_ref):
    o_ref[:] = jnp.dot(x_ref[:], w_ref[:],
                       preferred_element_type=jnp.float32) + b_ref[:]


def _update_body(p0_ref, p1_ref, h_ref, wr_ref, br_ref, wt_ref, o_ref):
    agg = p0_ref[:] + p1_ref[:]
    o_ref[:] = jnp.maximum(
        jnp.dot(agg, wr_ref[:], preferred_element_type=jnp.float32)
        + br_ref[:]
        + jnp.dot(h_ref[:], wt_ref[:], preferred_element_type=jnp.float32),
        0.0)


def _final_body(p0_ref, p1_ref, h_ref, wr_ref, br_ref, wt_ref,
                lnw_ref, lnb_ref, wf_ref, bf_ref, o_ref):
    agg = p0_ref[:] + p1_ref[:]
    h2 = jnp.maximum(
        jnp.dot(agg, wr_ref[:], preferred_element_type=jnp.float32)
        + br_ref[:]
        + jnp.dot(h_ref[:], wt_ref[:], preferred_element_type=jnp.float32),
        0.0)
    mu = jnp.mean(h2, axis=-1, keepdims=True)
    cent = h2 - mu
    var = jnp.mean(cent * cent, axis=-1, keepdims=True)
    normed = cent * lax.rsqrt(var + 1e-5) * lnw_ref[:] + lnb_ref[:]
    o_ref[:] = jnp.dot(normed, wf_ref[:],
                       preferred_element_type=jnp.float32) + bf_ref[:]


def _row_spec():
    return pl.BlockSpec((ROW_BLOCK, D), lambda i: (i, 0))


def _full_spec():
    return pl.BlockSpec((D, D), lambda i: (0, 0))


def _vec_spec():
    return pl.BlockSpec((1, D), lambda i: (0, 0))


def _tc_proj(x, w_t, b):
    n = x.shape[0]
    return pl.pallas_call(
        _proj_body,
        grid=(n // ROW_BLOCK,),
        in_specs=[_row_spec(), _full_spec(), _vec_spec()],
        out_specs=_row_spec(),
        out_shape=jax.ShapeDtypeStruct((n, D), jnp.float32),
    )(x, w_t, b)


def _tc_update(p0, p1, h, wr_t, br, wt_t):
    n = h.shape[0]
    return pl.pallas_call(
        _update_body,
        grid=(n // ROW_BLOCK,),
        in_specs=[_row_spec(), _row_spec(), _row_spec(),
                  _full_spec(), _vec_spec(), _full_spec()],
        out_specs=_row_spec(),
        out_shape=jax.ShapeDtypeStruct((n, D), jnp.float32),
    )(p0, p1, h, wr_t, br, wt_t)


def _tc_final(p0, p1, h, wr_t, br, wt_t, lnw, lnb, wf_t, bf):
    n = h.shape[0]
    return pl.pallas_call(
        _final_body,
        grid=(n // ROW_BLOCK,),
        in_specs=[_row_spec(), _row_spec(), _row_spec(),
                  _full_spec(), _vec_spec(), _full_spec(),
                  _vec_spec(), _vec_spec(), _full_spec(), _vec_spec()],
        out_specs=_row_spec(),
        out_shape=jax.ShapeDtypeStruct((n, D), jnp.float32),
    )(p0, p1, h, wr_t, br, wt_t, lnw, lnb, wf_t, bf)


def kernel(x, edge_index, Wp, bp, W_rel1, b_rel1, W_root1,
           W_rel2, b_rel2, W_root2, ln_w, ln_b, Wf, bf):
    n = x.shape[0]
    e = edge_index.shape[1]

    n_pad = -(-(n + 1) // 128) * 128
    ep = -(-e // (NW * CHUNK)) * (NW * CHUNK)

    src = edge_index[0].astype(jnp.int32)
    dst = edge_index[1].astype(jnp.int32)
    pad_e = ep - e
    if pad_e:
        ar = jnp.arange(pad_e, dtype=jnp.int32)
        src = jnp.concatenate([src, ar % n])
        dst = jnp.concatenate([dst, n + (ar % (n_pad - n))])
    zeros = jnp.zeros((n_pad, D), jnp.float32)

    wp_t = Wp.T
    wr1_t, wt1_t = W_rel1.T, W_root1.T
    wr2_t, wt2_t = W_rel2.T, W_root2.T
    wf_t = Wf.T
    bp2 = bp.reshape(1, D)
    br1 = b_rel1.reshape(1, D)
    br2 = b_rel2.reshape(1, D)
    lnw2 = ln_w.reshape(1, D)
    lnb2 = ln_b.reshape(1, D)
    bf2 = bf.reshape(1, D)

    h0 = _tc_proj(x, wp_t, bp2)

    part1 = _sc_segment_sum(h0, src, dst, zeros, n_pad=n_pad, ep=ep)
    h1 = _tc_update(part1[0, :n], part1[1, :n], h0, wr1_t, br1, wt1_t)

    part2 = _sc_segment_sum(h1, src, dst, zeros, n_pad=n_pad, ep=ep)
    out = _tc_final(part2[0, :n], part2[1, :n], h1, wr2_t, br2, wt2_t,
                    lnw2, lnb2, wf_t, bf2)
    return out


# speedup vs baseline: 10.6181x; 1.0462x over previous
import functools

import jax
import jax.numpy as jnp
from jax import lax
from jax.experimental import pallas as pl
from jax.experimental.pallas import tpu as pltpu
from jax.experimental.pallas import tpu_sc as plsc

D = 128
NW = 32
CHUNK = 128
ROW_BLOCK = 1000


@functools.partial(jax.jit, static_argnames=("n_pad", "ep"))
def _sc_segment_sum(h, idx, zeros, *, n_pad, ep):
    epw = ep // NW
    n_chunks = epw // CHUNK
    rows_per_tile = n_pad // 16

    mesh = plsc.VectorSubcoreMesh(core_axis_name="c", subcore_axis_name="s")

    @functools.partial(
        pl.kernel,
        mesh=mesh,
        out_type=jax.ShapeDtypeStruct((2, n_pad, D), jnp.float32),
        scratch_types=[
            pltpu.VMEM((2, 2, CHUNK), jnp.int32),
            pltpu.VMEM((2, CHUNK, D), jnp.float32),
            pltpu.VMEM_SHARED((n_pad, D), jnp.float32),
            pltpu.SemaphoreType.DMA,
            pltpu.SemaphoreType.DMA,
            pltpu.SemaphoreType.DMA,
            pltpu.SemaphoreType.DMA,
        ],
    )
    def agg(h_hbm, idx_hbm, zero_hbm, out_hbm,
            idx_v, rows_v, acc_sh, semi0, semi1, semg0, semg1):
        c = lax.axis_index("c")
        s = lax.axis_index("s")
        wid = c * 16 + s
        r0 = s * rows_per_tile
        pltpu.sync_copy(zero_hbm.at[pl.ds(r0, rows_per_tile)],
                        acc_sh.at[pl.ds(r0, rows_per_tile)])
        plsc.subcore_barrier()

        semi = (semi0, semi1)
        semg = (semg0, semg1)
        base0 = wid * epw

        def idx_copies(j, b):
            base = base0 + j * CHUNK
            return (
                pltpu.make_async_copy(idx_hbm.at[pl.ds(base, CHUNK)],
                                      idx_v.at[b].at[0], semi[b]),
                pltpu.make_async_copy(idx_hbm.at[pl.ds(ep + base, CHUNK)],
                                      idx_v.at[b].at[1], semi[b]),
            )

        def idx_load(j, b):
            class _Pair:
                def start(self):
                    for cp in idx_copies(j, b):
                        cp.start()

                def wait(self):
                    for cp in idx_copies(j, b):
                        cp.wait()
            return _Pair()

        def gather(j, b):
            del j
            return pltpu.make_async_copy(h_hbm.at[idx_v.at[b].at[0]],
                                         rows_v.at[b], semg[b])

        idx_load(0, 0).start()
        idx_load(1, 1).start()
        idx_load(0, 0).wait()
        gather(0, 0).start()

        def body(j, carry):
            b = lax.rem(j, 2)
            for bb in range(2):
                @pl.when(b == bb)
                def _():
                    gather(j, bb).wait()

                    @pl.when(j + 1 < n_chunks)
                    def _():
                        idx_load(j + 1, 1 - bb).wait()
                        gather(j + 1, 1 - bb).start()
                    pltpu.sync_copy(rows_v.at[bb],
                                    acc_sh.at[idx_v.at[bb].at[1]],
                                    add=True)

                    @pl.when(j + 2 < n_chunks)
                    def _():
                        idx_load(j + 2, bb).start()
            return carry

        lax.fori_loop(0, n_chunks, body, 0)
        plsc.subcore_barrier()
        pltpu.sync_copy(acc_sh.at[pl.ds(r0, rows_per_tile)],
                        out_hbm.at[c].at[pl.ds(r0, rows_per_tile)])

    return agg(h, idx, zeros)


def _proj_body(x_ref, w_ref, b_ref, o_ref):
    o_ref[:] = jnp.dot(x_ref[:], w_ref[:],
                       preferred_element_type=jnp.float32) + b_ref[:]


def _update_body(p_ref, h_ref, wr_ref, br_ref, wt_ref, o_ref):
    agg = p_ref[0] + p_ref[1]
    o_ref[:] = jnp.maximum(
        jnp.dot(agg, wr_ref[:], preferred_element_type=jnp.float32)
        + br_ref[:]
        + jnp.dot(h_ref[:], wt_ref[:], preferred_element_type=jnp.float32),
        0.0)


def _final_body(p_ref, h_ref, wr_ref, br_ref, wt_ref,
                lnw_ref, lnb_ref, wf_ref, bf_ref, o_ref):
    agg = p_ref[0] + p_ref[1]
    h2 = jnp.maximum(
        jnp.dot(agg, wr_ref[:], preferred_element_type=jnp.float32)
        + br_ref[:]
        + jnp.dot(h_ref[:], wt_ref[:], preferred_element_type=jnp.float32),
        0.0)
    mu = jnp.mean(h2, axis=-1, keepdims=True)
    cent = h2 - mu
    var = jnp.mean(cent * cent, axis=-1, keepdims=True)
    normed = cent * lax.rsqrt(var + 1e-5) * lnw_ref[:] + lnb_ref[:]
    o_ref[:] = jnp.dot(normed, wf_ref[:],
                       preferred_element_type=jnp.float32) + bf_ref[:]


def _row_spec():
    return pl.BlockSpec((ROW_BLOCK, D), lambda i: (i, 0))


def _full_spec():
    return pl.BlockSpec((D, D), lambda i: (0, 0))


def _vec_spec():
    return pl.BlockSpec((1, D), lambda i: (0, 0))


def _tc_proj(x, w_t, b):
    n = x.shape[0]
    return pl.pallas_call(
        _proj_body,
        grid=(n // ROW_BLOCK,),
        in_specs=[_row_spec(), _full_spec(), _vec_spec()],
        out_specs=_row_spec(),
        out_shape=jax.ShapeDtypeStruct((n, D), jnp.float32),
    )(x, w_t, b)


def _part_spec():
    return pl.BlockSpec((2, ROW_BLOCK, D), lambda i: (0, i, 0))


def _tc_update(part, h, wr_t, br, wt_t):
    n = h.shape[0]
    return pl.pallas_call(
        _update_body,
        grid=(n // ROW_BLOCK,),
        in_specs=[_part_spec(), _row_spec(),
                  _full_spec(), _vec_spec(), _full_spec()],
        out_specs=_row_spec(),
        out_shape=jax.ShapeDtypeStruct((n, D), jnp.float32),
    )(part, h, wr_t, br, wt_t)


def _tc_final(part, h, wr_t, br, wt_t, lnw, lnb, wf_t, bf):
    n = h.shape[0]
    return pl.pallas_call(
        _final_body,
        grid=(n // ROW_BLOCK,),
        in_specs=[_part_spec(), _row_spec(),
                  _full_spec(), _vec_spec(), _full_spec(),
                  _vec_spec(), _vec_spec(), _full_spec(), _vec_spec()],
        out_specs=_row_spec(),
        out_shape=jax.ShapeDtypeStruct((n, D), jnp.float32),
    )(part, h, wr_t, br, wt_t, lnw, lnb, wf_t, bf)


def kernel(x, edge_index, Wp, bp, W_rel1, b_rel1, W_root1,
           W_rel2, b_rel2, W_root2, ln_w, ln_b, Wf, bf):
    n = x.shape[0]
    e = edge_index.shape[1]

    n_pad = -(-(n + 1) // 128) * 128
    ep = -(-e // (NW * CHUNK)) * (NW * CHUNK)

    import numpy as np
    pad_e = ep - e
    ar = np.arange(pad_e, dtype=np.int32)
    ei = jnp.concatenate([
        edge_index[0].astype(jnp.int32), jnp.asarray(ar % n),
        edge_index[1].astype(jnp.int32),
        jnp.asarray(n + (ar % (n_pad - n))),
    ])
    zeros = jnp.zeros((n_pad, D), jnp.float32)

    wp_t = Wp.T
    wr1_t, wt1_t = W_rel1.T, W_root1.T
    wr2_t, wt2_t = W_rel2.T, W_root2.T
    wf_t = Wf.T
    bp2 = bp.reshape(1, D)
    br1 = b_rel1.reshape(1, D)
    br2 = b_rel2.reshape(1, D)
    lnw2 = ln_w.reshape(1, D)
    lnb2 = ln_b.reshape(1, D)
    bf2 = bf.reshape(1, D)

    h0 = _tc_proj(x, wp_t, bp2)

    part1 = _sc_segment_sum(h0, ei, zeros, n_pad=n_pad, ep=ep)
    h1 = _tc_update(part1, h0, wr1_t, br1, wt1_t)

    part2 = _sc_segment_sum(h1, ei, zeros, n_pad=n_pad, ep=ep)
    out = _tc_final(part2, h1, wr2_t, br2, wt2_t, lnw2, lnb2, wf_t, bf2)
    return out


# speedup vs baseline: 10.9798x; 1.0341x over previous
import functools

import jax
import jax.numpy as jnp
from jax import lax
from jax.experimental import pallas as pl
from jax.experimental.pallas import tpu as pltpu
from jax.experimental.pallas import tpu_sc as plsc

D = 128
NW = 32
CHUNK = 128
ROW_BLOCK = 1000


@functools.partial(jax.jit, static_argnames=("n_pad", "ep"))
def _sc_segment_sum(h, idx, zeros, *, n_pad, ep):
    epw = ep // NW
    n_chunks = epw // CHUNK
    tail = epw - n_chunks * CHUNK
    rows_per_tile = n_pad // 16

    mesh = plsc.VectorSubcoreMesh(core_axis_name="c", subcore_axis_name="s")

    scratch = [
        pltpu.VMEM((2, 2, CHUNK), jnp.int32),
        pltpu.VMEM((2, CHUNK, D), jnp.float32),
        pltpu.VMEM_SHARED((n_pad, D), jnp.float32),
        pltpu.SemaphoreType.DMA,
        pltpu.SemaphoreType.DMA,
        pltpu.SemaphoreType.DMA,
        pltpu.SemaphoreType.DMA,
    ]
    if tail:
        scratch += [pltpu.VMEM((2, tail), jnp.int32),
                    pltpu.VMEM((tail, D), jnp.float32)]

    @functools.partial(
        pl.kernel,
        mesh=mesh,
        out_type=jax.ShapeDtypeStruct((2, n_pad, D), jnp.float32),
        scratch_types=scratch,
    )
    def agg(h_hbm, idx_hbm, zero_hbm, out_hbm,
            idx_v, rows_v, acc_sh, semi0, semi1, semg0, semg1, *tail_v):
        c = lax.axis_index("c")
        s = lax.axis_index("s")
        wid = c * 16 + s
        r0 = s * rows_per_tile
        pltpu.sync_copy(zero_hbm.at[pl.ds(r0, rows_per_tile)],
                        acc_sh.at[pl.ds(r0, rows_per_tile)])
        plsc.subcore_barrier()

        semi = (semi0, semi1)
        semg = (semg0, semg1)
        base0 = wid * epw

        def idx_copies(j, b):
            base = base0 + j * CHUNK
            return (
                pltpu.make_async_copy(idx_hbm.at[pl.ds(base, CHUNK)],
                                      idx_v.at[b].at[0], semi[b]),
                pltpu.make_async_copy(idx_hbm.at[pl.ds(ep + base, CHUNK)],
                                      idx_v.at[b].at[1], semi[b]),
            )

        def idx_load(j, b):
            class _Pair:
                def start(self):
                    for cp in idx_copies(j, b):
                        cp.start()

                def wait(self):
                    for cp in idx_copies(j, b):
                        cp.wait()
            return _Pair()

        def gather(j, b):
            del j
            return pltpu.make_async_copy(h_hbm.at[idx_v.at[b].at[0]],
                                         rows_v.at[b], semg[b])

        idx_load(0, 0).start()
        idx_load(1, 1).start()
        idx_load(0, 0).wait()
        gather(0, 0).start()

        def body(j, carry):
            b = lax.rem(j, 2)
            for bb in range(2):
                @pl.when(b == bb)
                def _():
                    gather(j, bb).wait()

                    @pl.when(j + 1 < n_chunks)
                    def _():
                        idx_load(j + 1, 1 - bb).wait()
                        gather(j + 1, 1 - bb).start()
                    pltpu.sync_copy(rows_v.at[bb],
                                    acc_sh.at[idx_v.at[bb].at[1]],
                                    add=True)

                    @pl.when(j + 2 < n_chunks)
                    def _():
                        idx_load(j + 2, bb).start()
            return carry

        lax.fori_loop(0, n_chunks, body, 0)

        if tail:
            idxt_v, rowst_v = tail_v
            tbase = base0 + n_chunks * CHUNK
            cps = (pltpu.make_async_copy(idx_hbm.at[pl.ds(tbase, tail)],
                                         idxt_v.at[0], semi0),
                   pltpu.make_async_copy(idx_hbm.at[pl.ds(ep + tbase, tail)],
                                         idxt_v.at[1], semi0))
            for cp in cps:
                cp.start()
            for cp in cps:
                cp.wait()
            pltpu.async_copy(h_hbm.at[idxt_v.at[0]], rowst_v, semg0).wait()
            pltpu.sync_copy(rowst_v, acc_sh.at[idxt_v.at[1]], add=True)

        plsc.subcore_barrier()
        pltpu.sync_copy(acc_sh.at[pl.ds(r0, rows_per_tile)],
                        out_hbm.at[c].at[pl.ds(r0, rows_per_tile)])

    return agg(h, idx, zeros)


def _proj_body(x_ref, w_ref, b_ref, o_ref):
    o_ref[:] = jnp.dot(x_ref[:], w_ref[:],
                       preferred_element_type=jnp.float32) + b_ref[:]


def _update_body(p_ref, h_ref, wr_ref, br_ref, wt_ref, o_ref):
    agg = p_ref[0] + p_ref[1]
    o_ref[:] = jnp.maximum(
        jnp.dot(agg, wr_ref[:], preferred_element_type=jnp.float32)
        + br_ref[:]
        + jnp.dot(h_ref[:], wt_ref[:], preferred_element_type=jnp.float32),
        0.0)


def _final_body(p_ref, h_ref, wr_ref, br_ref, wt_ref,
                lnw_ref, lnb_ref, wf_ref, bf_ref, o_ref):
    agg = p_ref[0] + p_ref[1]
    h2 = jnp.maximum(
        jnp.dot(agg, wr_ref[:], preferred_element_type=jnp.float32)
        + br_ref[:]
        + jnp.dot(h_ref[:], wt_ref[:], preferred_element_type=jnp.float32),
        0.0)
    mu = jnp.mean(h2, axis=-1, keepdims=True)
    cent = h2 - mu
    var = jnp.mean(cent * cent, axis=-1, keepdims=True)
    normed = cent * lax.rsqrt(var + 1e-5) * lnw_ref[:] + lnb_ref[:]
    o_ref[:] = jnp.dot(normed, wf_ref[:],
                       preferred_element_type=jnp.float32) + bf_ref[:]


def _row_spec():
    return pl.BlockSpec((ROW_BLOCK, D), lambda i: (i, 0))


def _full_spec():
    return pl.BlockSpec((D, D), lambda i: (0, 0))


def _vec_spec():
    return pl.BlockSpec((1, D), lambda i: (0, 0))


def _tc_proj(x, w_t, b):
    n = x.shape[0]
    return pl.pallas_call(
        _proj_body,
        grid=(n // ROW_BLOCK,),
        in_specs=[_row_spec(), _full_spec(), _vec_spec()],
        out_specs=_row_spec(),
        out_shape=jax.ShapeDtypeStruct((n, D), jnp.float32),
    )(x, w_t, b)


def _part_spec():
    return pl.BlockSpec((2, ROW_BLOCK, D), lambda i: (0, i, 0))


def _tc_update(part, h, wr_t, br, wt_t):
    n = h.shape[0]
    return pl.pallas_call(
        _update_body,
        grid=(n // ROW_BLOCK,),
        in_specs=[_part_spec(), _row_spec(),
                  _full_spec(), _vec_spec(), _full_spec()],
        out_specs=_row_spec(),
        out_shape=jax.ShapeDtypeStruct((n, D), jnp.float32),
    )(part, h, wr_t, br, wt_t)


def _tc_final(part, h, wr_t, br, wt_t, lnw, lnb, wf_t, bf):
    n = h.shape[0]
    return pl.pallas_call(
        _final_body,
        grid=(n // ROW_BLOCK,),
        in_specs=[_part_spec(), _row_spec(),
                  _full_spec(), _vec_spec(), _full_spec(),
                  _vec_spec(), _vec_spec(), _full_spec(), _vec_spec()],
        out_specs=_row_spec(),
        out_shape=jax.ShapeDtypeStruct((n, D), jnp.float32),
    )(part, h, wr_t, br, wt_t, lnw, lnb, wf_t, bf)


def kernel(x, edge_index, Wp, bp, W_rel1, b_rel1, W_root1,
           W_rel2, b_rel2, W_root2, ln_w, ln_b, Wf, bf):
    n = x.shape[0]
    e = edge_index.shape[1]

    n_pad = -(-n // 128) * 128
    assert e % (8 * NW) == 0
    ei = edge_index.astype(jnp.int32).reshape(-1)
    zeros = jnp.zeros((n_pad, D), jnp.float32)

    wp_t = Wp.T
    wr1_t, wt1_t = W_rel1.T, W_root1.T
    wr2_t, wt2_t = W_rel2.T, W_root2.T
    wf_t = Wf.T
    bp2 = bp.reshape(1, D)
    br1 = b_rel1.reshape(1, D)
    br2 = b_rel2.reshape(1, D)
    lnw2 = ln_w.reshape(1, D)
    lnb2 = ln_b.reshape(1, D)
    bf2 = bf.reshape(1, D)

    h0 = _tc_proj(x, wp_t, bp2)

    part1 = _sc_segment_sum(h0, ei, zeros, n_pad=n_pad, ep=e)
    h1 = _tc_update(part1, h0, wr1_t, br1, wt1_t)

    part2 = _sc_segment_sum(h1, ei, zeros, n_pad=n_pad, ep=e)
    out = _tc_final(part2, h1, wr2_t, br2, wt2_t, lnw2, lnb2, wf_t, bf2)
    return out
